# Initial kernel scaffold; baseline (speedup 1.0000x reference)
import jax
import jax.numpy as jnp
from jax.experimental import pallas as pl


def kernel(h, x, edge_index, W1, b1, W2):
    raise NotImplementedError("write your pallas kernel here")


# speedup vs baseline: 1.7714x; 1.7714x over previous
import functools

import jax
import jax.numpy as jnp
from jax import lax
from jax.experimental import pallas as pl
from jax.experimental.pallas import tpu as pltpu
from jax.experimental.pallas import tpu_sc as plsc

DIM = 128
K = 32
WTEMP = 10.0
N_NODES = 10000
N_EDGES = 320000

NC, NS, L = 2, 16, 16
NW = NC * NS
E_PER_W = N_EDGES // NW
G = 80
NSUB = G // L
NGROUPS = E_PER_W // G


def _proj_body(h_ref, wa_ref, wb_ref, b_ref, a_ref, bo_ref):
    hblk = h_ref[...]
    a_ref[...] = jnp.dot(hblk, wa_ref[...],
                         preferred_element_type=jnp.float32) + b_ref[...]
    bo_ref[...] = jnp.dot(hblk, wb_ref[...],
                          preferred_element_type=jnp.float32)


def _node_proj(h, wa_t, wb_t, b1):
    BM = 2000
    grid = (N_NODES // BM,)
    return pl.pallas_call(
        _proj_body,
        grid=grid,
        in_specs=[
            pl.BlockSpec((BM, DIM), lambda i: (i, 0)),
            pl.BlockSpec((DIM, DIM), lambda i: (0, 0)),
            pl.BlockSpec((DIM, DIM), lambda i: (0, 0)),
            pl.BlockSpec((1, DIM), lambda i: (0, 0)),
        ],
        out_specs=[
            pl.BlockSpec((BM, DIM), lambda i: (i, 0)),
            pl.BlockSpec((BM, DIM), lambda i: (i, 0)),
        ],
        out_shape=[
            jax.ShapeDtypeStruct((N_NODES, DIM), jnp.float32),
            jax.ShapeDtypeStruct((N_NODES, DIM), jnp.float32),
        ],
    )(h, wa_t, wb_t, b1.reshape(1, DIM))


def _reduce_body(p_ref, o_ref):
    o_ref[...] = jnp.sum(p_ref[...], axis=0, keepdims=True) * (1.0 / K)


def _reduce_partials(partials):
    return pl.pallas_call(
        _reduce_body,
        out_shape=jax.ShapeDtypeStruct((1, 3 * N_NODES), jnp.float32),
    )(partials)


def _rsqrt_f32(a):
    i = plsc.bitcast(a, jnp.int32)
    i = jnp.int32(0x5F3759DF) - lax.shift_right_arithmetic(i, 1)
    y = plsc.bitcast(i, jnp.float32)
    h = a * 0.5
    for _ in range(3):
        y = y * (1.5 - h * y * y)
    return y


def _edge_body(a_hbm, b_hbm, x_hbm, row_hbm, col_hbm, wc_hbm, w2_hbm,
               zero_hbm, out_hbm,
               x_v, wc_v, w2_v, idx_a, idx_b, a_buf, b_buf, vacc, sem_a,
               sem_b):
    cid = lax.axis_index("c")
    sid = lax.axis_index("s")
    wid = sid * NC + cid
    tile_base = wid * E_PER_W

    pltpu.sync_copy(x_hbm, x_v)
    pltpu.sync_copy(wc_hbm, wc_v)
    pltpu.sync_copy(w2_hbm, w2_v)
    pltpu.sync_copy(zero_hbm, vacc)

    lanes = [lax.iota(jnp.int32, 16) + (16 * s) for s in range(NSUB)]

    def group_body(g, _):
        base = tile_base + g * G
        pltpu.sync_copy(row_hbm.at[pl.ds(base, G)], idx_a)
        pltpu.sync_copy(col_hbm.at[pl.ds(base, G)], idx_b)
        cp_a = pltpu.async_copy(a_hbm.at[idx_a], a_buf, sem_a)
        cp_b = pltpu.async_copy(b_hbm.at[idx_b], b_buf, sem_b)

        r3 = []
        dsqs = []
        for s in range(NSUB):
            row_s = idx_a[pl.ds(16 * s, 16)]
            col_s = idx_b[pl.ds(16 * s, 16)]
            r3_s = row_s * 3
            c3_s = col_s * 3
            dsq = jnp.zeros((16,), jnp.float32)
            for c in range(3):
                rij = (plsc.load_gather(x_v, [r3_s + c])
                       - plsc.load_gather(x_v, [c3_s + c]))
                dsq = dsq + rij * rij
            r3.append(r3_s)
            dsqs.append(dsq)

        cp_a.wait()
        cp_b.wait()

        def j_body(jo, accs):
            wcv = wc_v[pl.ds(jo * 16, 16)]
            w2v = w2_v[pl.ds(jo * 16, 16)]
            jbase = jo * 16
            out = list(accs)
            for ji in range(16):
                wcj = wcv[ji]
                w2j = w2v[ji]
                jf = jnp.full((16,), 0, dtype=jnp.int32) + (jbase + ji)
                for s in range(NSUB):
                    a = plsc.load_gather(a_buf, [lanes[s], jf])
                    b = plsc.load_gather(b_buf, [lanes[s], jf])
                    z = a + b + dsqs[s] * wcj
                    sil = z / (1.0 + jnp.exp(-z))
                    out[s] = out[s] + w2j * sil
            return tuple(out)

        accs = lax.fori_loop(0, DIM // 16, j_body,
                             tuple(jnp.zeros((16,), jnp.float32)
                                   for _ in range(NSUB)))

        for s in range(NSUB):
            u = accs[s] * (1.0 / WTEMP)
            t = jnp.exp(-2.0 * jnp.abs(u))
            th = (1.0 - t) / (1.0 + t)
            w = jnp.where(u < 0.0, -th, th)
            rs = _rsqrt_f32(dsqs[s] + 1e-8)
            wrs = w * rs
            r3_s = r3[s]
            c3_s = idx_b[pl.ds(16 * s, 16)] * 3
            for c in range(3):
                rij = (plsc.load_gather(x_v, [r3_s + c])
                       - plsc.load_gather(x_v, [c3_s + c]))
                plsc.addupdate_scatter(vacc, [r3_s + c], wrs * rij)
        return 0

    lax.fori_loop(0, NGROUPS, group_body, 0)
    pltpu.sync_copy(vacc, out_hbm.at[wid])


def _edge_pass(a, b, x_flat, row, col, wc, w2, zero):
    mesh = plsc.VectorSubcoreMesh(core_axis_name="c", subcore_axis_name="s",
                                  num_cores=NC, num_subcores=NS)
    kern = pl.kernel(
        _edge_body,
        out_type=jax.ShapeDtypeStruct((NW, 3 * N_NODES), jnp.float32),
        mesh=mesh,
        compiler_params=pltpu.CompilerParams(needs_layout_passes=False),
        scratch_types=[
            pltpu.VMEM((3 * N_NODES,), jnp.float32),
            pltpu.VMEM((DIM,), jnp.float32),
            pltpu.VMEM((DIM,), jnp.float32),
            pltpu.VMEM((G,), jnp.int32),
            pltpu.VMEM((G,), jnp.int32),
            pltpu.VMEM((G, DIM), jnp.float32),
            pltpu.VMEM((G, DIM), jnp.float32),
            pltpu.VMEM((3 * N_NODES,), jnp.float32),
            pltpu.SemaphoreType.DMA,
            pltpu.SemaphoreType.DMA,
        ],
    )
    return kern(a, b, x_flat, row, col, wc, w2, zero)


def kernel(h, x, edge_index, W1, b1, W2):
    row = edge_index[0].astype(jnp.int32)
    col = edge_index[1].astype(jnp.int32)
    wa_t = W1[:, :DIM].T
    wb_t = W1[:, DIM:2 * DIM].T
    wc = W1[:, 2 * DIM]
    w2 = W2.reshape(DIM)
    x_flat = x.reshape(-1)
    zero = jnp.zeros((3 * N_NODES,), jnp.float32)

    a, b = _node_proj(h, wa_t, wb_t, b1)
    partials = _edge_pass(a, b, x_flat, row, col, wc, w2, zero)
    v = _reduce_partials(partials)
    return v.reshape(N_NODES, 3)


# speedup vs baseline: 2.9104x; 1.6430x over previous
import functools

import jax
import jax.numpy as jnp
from jax import lax
from jax.experimental import pallas as pl
from jax.experimental.pallas import tpu as pltpu
from jax.experimental.pallas import tpu_sc as plsc

DIM = 128
K = 32
WTEMP = 10.0
N_NODES = 10000
N_EDGES = 320000

NC, NS, L = 2, 16, 16
NW = NC * NS
E_PER_W = N_EDGES // NW
G = 80
NSUB = G // L
NGROUPS = E_PER_W // G


def _proj_body(h_ref, wa_ref, wb_ref, b_ref, a_ref, bo_ref):
    hblk = h_ref[...]
    a_ref[...] = jnp.dot(hblk, wa_ref[...],
                         preferred_element_type=jnp.float32) + b_ref[...]
    bo_ref[...] = jnp.dot(hblk, wb_ref[...],
                          preferred_element_type=jnp.float32)


def _node_proj(h, wa_t, wb_t, b1):
    BM = 2000
    grid = (N_NODES // BM,)
    return pl.pallas_call(
        _proj_body,
        grid=grid,
        in_specs=[
            pl.BlockSpec((BM, DIM), lambda i: (i, 0)),
            pl.BlockSpec((DIM, DIM), lambda i: (0, 0)),
            pl.BlockSpec((DIM, DIM), lambda i: (0, 0)),
            pl.BlockSpec((1, DIM), lambda i: (0, 0)),
        ],
        out_specs=[
            pl.BlockSpec((BM, DIM), lambda i: (i, 0)),
            pl.BlockSpec((BM, DIM), lambda i: (i, 0)),
        ],
        out_shape=[
            jax.ShapeDtypeStruct((N_NODES, DIM), jnp.float32),
            jax.ShapeDtypeStruct((N_NODES, DIM), jnp.float32),
        ],
    )(h, wa_t, wb_t, b1.reshape(1, DIM))


def _reduce_body(p_ref, o_ref):
    o_ref[...] = jnp.sum(p_ref[...], axis=0, keepdims=True) * (1.0 / K)


def _reduce_partials(partials):
    return pl.pallas_call(
        _reduce_body,
        out_shape=jax.ShapeDtypeStruct((1, 3 * N_NODES), jnp.float32),
    )(partials)


def _rsqrt_f32(a):
    i = plsc.bitcast(a, jnp.int32)
    i = jnp.int32(0x5F3759DF) - lax.shift_right_arithmetic(i, 1)
    y = plsc.bitcast(i, jnp.float32)
    h = a * 0.5
    for _ in range(3):
        y = y * (1.5 - h * y * y)
    return y


def _edge_body(a_hbm, b_hbm, x_hbm, row_hbm, col_hbm, wc_hbm, w2_hbm,
               zero_hbm, out_hbm,
               x_v, wc_v, w2_v, row_v, col_v, zb0, zb1, zb2, vacc,
               sa0, sa1, sa2, sb0, sb1, sb2):
    cid = lax.axis_index("c")
    sid = lax.axis_index("s")
    wid = sid * NC + cid
    tile_base = wid * E_PER_W

    pltpu.sync_copy(x_hbm, x_v)
    pltpu.sync_copy(wc_hbm, wc_v)
    pltpu.sync_copy(w2_hbm, w2_v)
    pltpu.sync_copy(row_hbm.at[pl.ds(tile_base, E_PER_W)], row_v)
    pltpu.sync_copy(col_hbm.at[pl.ds(tile_base, E_PER_W)], col_v)
    pltpu.sync_copy(zero_hbm, vacc)

    lanes = [lax.iota(jnp.int32, 16) + (16 * s) for s in range(NSUB)]
    bufs = (zb0, zb1, zb2)
    sas = (sa0, sa1, sa2)
    sbs = (sb0, sb1, sb2)

    def issue_a(g, p):
        pltpu.async_copy(a_hbm.at[row_v.at[pl.ds(g * G, G)]], bufs[p],
                         sas[p])

    def issue_b(g, p):
        pltpu.async_copy(b_hbm.at[col_v.at[pl.ds(g * G, G)]], bufs[p],
                         sbs[p], add=True)

    def wait(sem, buf):
        pltpu.make_async_copy(a_hbm.at[pl.ds(0, G)], buf, sem).wait()

    issue_a(0, 0)
    wait(sas[0], bufs[0])
    issue_b(0, 0)
    issue_a(1, 1)

    def process(g, p):
        @pl.when(g + 1 < NGROUPS)
        def _():
            wait(sas[(p + 1) % 3], bufs[(p + 1) % 3])
            issue_b(g + 1, (p + 1) % 3)

        @pl.when(g + 2 < NGROUPS)
        def _():
            issue_a(g + 2, (p + 2) % 3)

        r3 = []
        dsqs = []
        rijs = []
        for s in range(NSUB):
            row_s = row_v[pl.ds(g * G + 16 * s, 16)]
            col_s = col_v[pl.ds(g * G + 16 * s, 16)]
            r3_s = row_s * 3
            c3_s = col_s * 3
            dsq = jnp.zeros((16,), jnp.float32)
            rij_s = []
            for c in range(3):
                rij = (plsc.load_gather(x_v, [r3_s + c])
                       - plsc.load_gather(x_v, [c3_s + c]))
                dsq = dsq + rij * rij
                rij_s.append(rij)
            r3.append(r3_s)
            dsqs.append(dsq)
            rijs.append(rij_s)

        wait(sbs[p], bufs[p])
        zbuf = bufs[p]

        def j_body(jo, accs):
            wcv = wc_v[pl.ds(jo * 16, 16)]
            w2v = w2_v[pl.ds(jo * 16, 16)]
            jbase = jo * 16
            out = list(accs)
            for ji in range(16):
                wcj = wcv[ji]
                w2j = w2v[ji]
                jf = jnp.full((16,), 0, dtype=jnp.int32) + (jbase + ji)
                for s in range(NSUB):
                    z = plsc.load_gather(zbuf, [lanes[s], jf]) + dsqs[s] * wcj
                    sil = z / (1.0 + jnp.exp(-z))
                    out[s] = out[s] + w2j * sil
            return tuple(out)

        accs = lax.fori_loop(0, DIM // 16, j_body,
                             tuple(jnp.zeros((16,), jnp.float32)
                                   for _ in range(NSUB)))

        for s in range(NSUB):
            u = accs[s] * (1.0 / WTEMP)
            t = jnp.exp(-2.0 * jnp.abs(u))
            th = (1.0 - t) / (1.0 + t)
            w = jnp.where(u < 0.0, -th, th)
            rs = _rsqrt_f32(dsqs[s] + 1e-8)
            wrs = w * rs
            for c in range(3):
                plsc.addupdate_scatter(vacc, [r3[s] + c], wrs * rijs[s][c])

    def group_body(g3, _):
        for p in range(3):
            g = g3 * 3 + p

            @pl.when(g < NGROUPS)
            def _():
                process(g, p)
        return 0

    lax.fori_loop(0, (NGROUPS + 2) // 3, group_body, 0)
    pltpu.sync_copy(vacc, out_hbm.at[wid])


def _edge_pass(a, b, x_flat, row, col, wc, w2, zero):
    mesh = plsc.VectorSubcoreMesh(core_axis_name="c", subcore_axis_name="s",
                                  num_cores=NC, num_subcores=NS)
    kern = pl.kernel(
        _edge_body,
        out_type=jax.ShapeDtypeStruct((NW, 3 * N_NODES), jnp.float32),
        mesh=mesh,
        compiler_params=pltpu.CompilerParams(needs_layout_passes=False),
        scratch_types=[
            pltpu.VMEM((3 * N_NODES,), jnp.float32),
            pltpu.VMEM((DIM,), jnp.float32),
            pltpu.VMEM((DIM,), jnp.float32),
            pltpu.VMEM((E_PER_W,), jnp.int32),
            pltpu.VMEM((E_PER_W,), jnp.int32),
            pltpu.VMEM((G, DIM), jnp.float32),
            pltpu.VMEM((G, DIM), jnp.float32),
            pltpu.VMEM((G, DIM), jnp.float32),
            pltpu.VMEM((3 * N_NODES,), jnp.float32),
            pltpu.SemaphoreType.DMA,
            pltpu.SemaphoreType.DMA,
            pltpu.SemaphoreType.DMA,
            pltpu.SemaphoreType.DMA,
            pltpu.SemaphoreType.DMA,
            pltpu.SemaphoreType.DMA,
        ],
    )
    return kern(a, b, x_flat, row, col, wc, w2, zero)


def kernel(h, x, edge_index, W1, b1, W2):
    row = edge_index[0].astype(jnp.int32)
    col = edge_index[1].astype(jnp.int32)
    wa_t = W1[:, :DIM].T
    wb_t = W1[:, DIM:2 * DIM].T
    wc = W1[:, 2 * DIM]
    w2 = W2.reshape(DIM)
    x_flat = x.reshape(-1)
    zero = jnp.zeros((3 * N_NODES,), jnp.float32)

    a, b = _node_proj(h, wa_t, wb_t, b1)
    partials = _edge_pass(a, b, x_flat, row, col, wc, w2, zero)
    v = _reduce_partials(partials)
    return v.reshape(N_NODES, 3)


# speedup vs baseline: 4.5221x; 1.5538x over previous
import functools

import jax
import jax.numpy as jnp
from jax import lax
from jax.experimental import pallas as pl
from jax.experimental.pallas import tpu as pltpu
from jax.experimental.pallas import tpu_sc as plsc

DIM = 128
K = 32
WTEMP = 10.0
N_NODES = 10000
N_EDGES = 320000

NC, NS, L = 2, 16, 16
NW = NC * NS
E_PER_W = N_EDGES // NW
G = 80
NSUB = G // L
NGROUPS = E_PER_W // G


def _proj_body(h_ref, wa_ref, wb_ref, b_ref, a_ref, bo_ref):
    hblk = h_ref[...]
    a_ref[...] = jnp.dot(hblk, wa_ref[...],
                         preferred_element_type=jnp.float32) + b_ref[...]
    bo_ref[...] = jnp.dot(hblk, wb_ref[...],
                          preferred_element_type=jnp.float32)


def _node_proj(h, wa_t, wb_t, b1):
    BM = 2000
    grid = (N_NODES // BM,)
    return pl.pallas_call(
        _proj_body,
        grid=grid,
        in_specs=[
            pl.BlockSpec((BM, DIM), lambda i: (i, 0)),
            pl.BlockSpec((DIM, DIM), lambda i: (0, 0)),
            pl.BlockSpec((DIM, DIM), lambda i: (0, 0)),
            pl.BlockSpec((1, DIM), lambda i: (0, 0)),
        ],
        out_specs=[
            pl.BlockSpec((BM, DIM), lambda i: (i, 0)),
            pl.BlockSpec((BM, DIM), lambda i: (i, 0)),
        ],
        out_shape=[
            jax.ShapeDtypeStruct((N_NODES, DIM), jnp.float32),
            jax.ShapeDtypeStruct((N_NODES, DIM), jnp.float32),
        ],
    )(h, wa_t, wb_t, b1.reshape(1, DIM))


def _reduce_body(p_ref, o_ref):
    o_ref[...] = jnp.sum(p_ref[...], axis=0, keepdims=True) * (1.0 / K)


def _reduce_partials(partials):
    return pl.pallas_call(
        _reduce_body,
        out_shape=jax.ShapeDtypeStruct((1, 3 * N_NODES), jnp.float32),
    )(partials)


def _rsqrt_f32(a):
    i = plsc.bitcast(a, jnp.int32)
    i = jnp.int32(0x5F3759DF) - lax.shift_right_arithmetic(i, 1)
    y = plsc.bitcast(i, jnp.float32)
    h = a * 0.5
    for _ in range(3):
        y = y * (1.5 - h * y * y)
    return y


LUT_N = 4096
LUT_SCALE = 128.0
LUT_OFF = float(LUT_N // 2)


def _edge_body(a_hbm, b_hbm, x_hbm, row_hbm, col_hbm, wc_hbm, w2_hbm,
               zero_hbm, lut_hbm, out_hbm,
               x_v, wc_v, w2_v, row_v, col_v, lut_v, zb0, zb1, zb2, vacc,
               sa0, sa1, sa2, sb0, sb1, sb2):
    cid = lax.axis_index("c")
    sid = lax.axis_index("s")
    wid = sid * NC + cid
    tile_base = wid * E_PER_W

    pltpu.sync_copy(x_hbm, x_v)
    pltpu.sync_copy(wc_hbm, wc_v)
    pltpu.sync_copy(w2_hbm, w2_v)
    pltpu.sync_copy(row_hbm.at[pl.ds(tile_base, E_PER_W)], row_v)
    pltpu.sync_copy(col_hbm.at[pl.ds(tile_base, E_PER_W)], col_v)
    pltpu.sync_copy(lut_hbm, lut_v)
    pltpu.sync_copy(zero_hbm, vacc)

    lanes = [lax.iota(jnp.int32, 16) + (16 * s) for s in range(NSUB)]
    bufs = (zb0, zb1, zb2)
    sas = (sa0, sa1, sa2)
    sbs = (sb0, sb1, sb2)

    def issue_a(g, p):
        pltpu.async_copy(a_hbm.at[row_v.at[pl.ds(g * G, G)]], bufs[p],
                         sas[p])

    def issue_b(g, p):
        pltpu.async_copy(b_hbm.at[col_v.at[pl.ds(g * G, G)]], bufs[p],
                         sbs[p], add=True)

    def wait(sem, buf):
        pltpu.make_async_copy(a_hbm.at[pl.ds(0, G)], buf, sem).wait()

    issue_a(0, 0)
    wait(sas[0], bufs[0])
    issue_b(0, 0)
    issue_a(1, 1)

    def process(g, p):
        @pl.when(g + 1 < NGROUPS)
        def _():
            wait(sas[(p + 1) % 3], bufs[(p + 1) % 3])
            issue_b(g + 1, (p + 1) % 3)

        @pl.when(g + 2 < NGROUPS)
        def _():
            issue_a(g + 2, (p + 2) % 3)

        r3 = []
        dsqs = []
        rijs = []
        for s in range(NSUB):
            row_s = row_v[pl.ds(g * G + 16 * s, 16)]
            col_s = col_v[pl.ds(g * G + 16 * s, 16)]
            r3_s = row_s * 3
            c3_s = col_s * 3
            dsq = jnp.zeros((16,), jnp.float32)
            rij_s = []
            for c in range(3):
                rij = (plsc.load_gather(x_v, [r3_s + c])
                       - plsc.load_gather(x_v, [c3_s + c]))
                dsq = dsq + rij * rij
                rij_s.append(rij)
            r3.append(r3_s)
            dsqs.append(dsq)
            rijs.append(rij_s)

        wait(sbs[p], bufs[p])
        zbuf = bufs[p]

        def j_body(jo, accs):
            wcv = wc_v[pl.ds(jo * 16, 16)]
            w2v = w2_v[pl.ds(jo * 16, 16)]
            jbase = jo * 16
            out = list(accs)
            for ji in range(16):
                wcj = wcv[ji]
                w2j = w2v[ji]
                jf = jnp.full((16,), 0, dtype=jnp.int32) + (jbase + ji)
                for s in range(NSUB):
                    z = plsc.load_gather(zbuf, [lanes[s], jf]) + dsqs[s] * wcj
                    t = z * LUT_SCALE + LUT_OFF
                    t = jnp.minimum(jnp.maximum(t, 0.0), float(LUT_N - 1))
                    sg = plsc.load_gather(lut_v, [t.astype(jnp.int32)])
                    out[s] = out[s] + w2j * (z * sg)
            return tuple(out)

        accs = lax.fori_loop(0, DIM // 16, j_body,
                             tuple(jnp.zeros((16,), jnp.float32)
                                   for _ in range(NSUB)))

        for s in range(NSUB):
            u = accs[s] * (1.0 / WTEMP)
            t = jnp.exp(-2.0 * jnp.abs(u))
            th = (1.0 - t) / (1.0 + t)
            w = jnp.where(u < 0.0, -th, th)
            rs = _rsqrt_f32(dsqs[s] + 1e-8)
            wrs = w * rs
            for c in range(3):
                plsc.addupdate_scatter(vacc, [r3[s] + c], wrs * rijs[s][c])

    def group_body(g3, _):
        for p in range(3):
            g = g3 * 3 + p

            @pl.when(g < NGROUPS)
            def _():
                process(g, p)
        return 0

    lax.fori_loop(0, (NGROUPS + 2) // 3, group_body, 0)
    pltpu.sync_copy(vacc, out_hbm.at[wid])


def _edge_pass(a, b, x_flat, row, col, wc, w2, zero, lut):
    mesh = plsc.VectorSubcoreMesh(core_axis_name="c", subcore_axis_name="s",
                                  num_cores=NC, num_subcores=NS)
    kern = pl.kernel(
        _edge_body,
        out_type=jax.ShapeDtypeStruct((NW, 3 * N_NODES), jnp.float32),
        mesh=mesh,
        compiler_params=pltpu.CompilerParams(needs_layout_passes=False),
        scratch_types=[
            pltpu.VMEM((3 * N_NODES,), jnp.float32),
            pltpu.VMEM((DIM,), jnp.float32),
            pltpu.VMEM((DIM,), jnp.float32),
            pltpu.VMEM((E_PER_W,), jnp.int32),
            pltpu.VMEM((E_PER_W,), jnp.int32),
            pltpu.VMEM((LUT_N,), jnp.float32),
            pltpu.VMEM((G, DIM), jnp.float32),
            pltpu.VMEM((G, DIM), jnp.float32),
            pltpu.VMEM((G, DIM), jnp.float32),
            pltpu.VMEM((3 * N_NODES,), jnp.float32),
            pltpu.SemaphoreType.DMA,
            pltpu.SemaphoreType.DMA,
            pltpu.SemaphoreType.DMA,
            pltpu.SemaphoreType.DMA,
            pltpu.SemaphoreType.DMA,
            pltpu.SemaphoreType.DMA,
        ],
    )
    return kern(a, b, x_flat, row, col, wc, w2, zero, lut)


def kernel(h, x, edge_index, W1, b1, W2):
    row = edge_index[0].astype(jnp.int32)
    col = edge_index[1].astype(jnp.int32)
    wa_t = W1[:, :DIM].T
    wb_t = W1[:, DIM:2 * DIM].T
    wc = W1[:, 2 * DIM]
    w2 = W2.reshape(DIM)
    x_flat = x.reshape(-1)
    zero = jnp.zeros((3 * N_NODES,), jnp.float32)
    lut = jax.nn.sigmoid(
        (jnp.arange(LUT_N, dtype=jnp.float32) - LUT_OFF) * (1.0 / LUT_SCALE))

    a, b = _node_proj(h, wa_t, wb_t, b1)
    partials = _edge_pass(a, b, x_flat, row, col, wc, w2, zero, lut)
    v = _reduce_partials(partials)
    return v.reshape(N_NODES, 3)


# speedup vs baseline: 11.3360x; 2.5068x over previous
import functools

import jax
import jax.numpy as jnp
from jax import lax
from jax.experimental import pallas as pl
from jax.experimental.pallas import tpu as pltpu
from jax.experimental.pallas import tpu_sc as plsc

DIM = 128
K = 32
WTEMP = 10.0
N_NODES = 10000
N_EDGES = 320000

NC, NS, L = 2, 16, 16
NW = NC * NS
E_PER_W = N_EDGES // NW
G = 80
NSUB = G // L
NGROUPS = E_PER_W // G


def _proj_body(h_ref, wa_ref, wb_ref, b_ref, a_ref, bo_ref):
    hblk = h_ref[...]
    a_ref[...] = jnp.dot(hblk, wa_ref[...],
                         preferred_element_type=jnp.float32) + b_ref[...]
    bo_ref[...] = jnp.dot(hblk, wb_ref[...],
                          preferred_element_type=jnp.float32) + LUT_OFF * 0.5


def _node_proj(h, wa_t, wb_t, b1):
    BM = 2000
    grid = (N_NODES // BM,)
    return pl.pallas_call(
        _proj_body,
        grid=grid,
        in_specs=[
            pl.BlockSpec((BM, DIM), lambda i: (i, 0)),
            pl.BlockSpec((DIM, DIM), lambda i: (0, 0)),
            pl.BlockSpec((DIM, DIM), lambda i: (0, 0)),
            pl.BlockSpec((1, DIM), lambda i: (0, 0)),
        ],
        out_specs=[
            pl.BlockSpec((BM, DIM), lambda i: (i, 0)),
            pl.BlockSpec((BM, DIM), lambda i: (i, 0)),
        ],
        out_shape=[
            jax.ShapeDtypeStruct((N_NODES, DIM), jnp.float32),
            jax.ShapeDtypeStruct((N_NODES, DIM), jnp.float32),
        ],
    )(h, wa_t, wb_t, b1.reshape(1, DIM))


def _reduce_body(p_ref, o_ref):
    o_ref[...] = jnp.sum(p_ref[...], axis=0, keepdims=True) * (1.0 / K)


def _reduce_partials(partials):
    return pl.pallas_call(
        _reduce_body,
        out_shape=jax.ShapeDtypeStruct((1, 3 * N_NODES), jnp.float32),
    )(partials)


def _rsqrt_f32(a):
    i = plsc.bitcast(a, jnp.int32)
    i = jnp.int32(0x5F3759DF) - lax.shift_right_arithmetic(i, 1)
    y = plsc.bitcast(i, jnp.float32)
    h = a * 0.5
    for _ in range(3):
        y = y * (1.5 - h * y * y)
    return y


LUT_N = 4096
LUT_SCALE = 128.0
LUT_OFF = float(LUT_N // 2)


def _edge_body(a_hbm, b_hbm, x_hbm, row_hbm, col_hbm, wc_hbm, w2_hbm,
               zero_hbm, lut_hbm, out_hbm,
               x_v, wc_v, w2_v, row_v, col_v, lut_v, zb0, zb1, zb2, vacc,
               sa0, sa1, sa2, sb0, sb1, sb2):
    cid = lax.axis_index("c")
    sid = lax.axis_index("s")
    wid = sid * NC + cid
    tile_base = wid * E_PER_W

    pltpu.sync_copy(x_hbm, x_v)
    pltpu.sync_copy(wc_hbm, wc_v)
    pltpu.sync_copy(w2_hbm, w2_v)
    pltpu.sync_copy(row_hbm.at[pl.ds(tile_base, E_PER_W)], row_v)
    pltpu.sync_copy(col_hbm.at[pl.ds(tile_base, E_PER_W)], col_v)
    pltpu.sync_copy(lut_hbm, lut_v)
    pltpu.sync_copy(zero_hbm, vacc)

    lanes = [lax.iota(jnp.int32, 16) + (16 * s) for s in range(NSUB)]
    bufs = (zb0, zb1, zb2)
    sas = (sa0, sa1, sa2)
    sbs = (sb0, sb1, sb2)

    def issue_a(g, p):
        pltpu.async_copy(a_hbm.at[row_v.at[pl.ds(g * G, G)]], bufs[p],
                         sas[p])

    def issue_b(g, p):
        pltpu.async_copy(b_hbm.at[col_v.at[pl.ds(g * G, G)]], bufs[p],
                         sbs[p], add=True)

    def wait(sem, buf):
        pltpu.make_async_copy(a_hbm.at[pl.ds(0, G)], buf, sem).wait()

    issue_a(0, 0)
    wait(sas[0], bufs[0])
    issue_b(0, 0)
    issue_a(1, 1)

    def process(g, p):
        @pl.when(g + 1 < NGROUPS)
        def _():
            wait(sas[(p + 1) % 3], bufs[(p + 1) % 3])
            issue_b(g + 1, (p + 1) % 3)

        @pl.when(g + 2 < NGROUPS)
        def _():
            issue_a(g + 2, (p + 2) % 3)

        r3 = []
        dsqs = []
        rijs = []
        for s in range(NSUB):
            row_s = row_v[pl.ds(g * G + 16 * s, 16)]
            col_s = col_v[pl.ds(g * G + 16 * s, 16)]
            r3_s = row_s * 3
            c3_s = col_s * 3
            dsq = jnp.zeros((16,), jnp.float32)
            rij_s = []
            for c in range(3):
                rij = (plsc.load_gather(x_v, [r3_s + c])
                       - plsc.load_gather(x_v, [c3_s + c]))
                dsq = dsq + rij * rij
                rij_s.append(rij)
            r3.append(r3_s)
            dsqs.append(dsq)
            rijs.append(rij_s)

        wait(sbs[p], bufs[p])
        zbuf = bufs[p]

        iota16 = lax.iota(jnp.int32, 16)

        def j_body(jo, accs):
            jbase = jo * 16
            out = list(accs)
            for ji in range(16):
                j = jbase + ji
                wcv = wc_v[pl.ds(j * 16, 16)]
                w2v = w2_v[pl.ds(j * 16, 16)]
                cid = (iota16 + j) & (DIM - 1)
                for s in range(NSUB):
                    t = (plsc.load_gather(zbuf, [lanes[s], cid])
                         + dsqs[s] * wcv)
                    tc = jnp.minimum(jnp.maximum(t, 0.0), float(LUT_N - 1))
                    sg = plsc.load_gather(lut_v, [tc.astype(jnp.int32)])
                    out[s] = out[s] + w2v * ((t - LUT_OFF) * sg)
            return tuple(out)

        accs = lax.fori_loop(0, DIM // 16, j_body,
                             tuple(jnp.zeros((16,), jnp.float32)
                                   for _ in range(NSUB)))

        for s in range(NSUB):
            u = accs[s] * (1.0 / WTEMP)
            t = jnp.exp(-2.0 * jnp.abs(u))
            th = (1.0 - t) / (1.0 + t)
            w = jnp.where(u < 0.0, -th, th)
            rs = _rsqrt_f32(dsqs[s] + 1e-8)
            wrs = w * rs
            for c in range(3):
                plsc.addupdate_scatter(vacc, [r3[s] + c], wrs * rijs[s][c])

    def group_body(g3, _):
        for p in range(3):
            g = g3 * 3 + p

            @pl.when(g < NGROUPS)
            def _():
                process(g, p)
        return 0

    lax.fori_loop(0, (NGROUPS + 2) // 3, group_body, 0)
    pltpu.sync_copy(vacc, out_hbm.at[wid])


def _edge_pass(a, b, x_flat, row, col, wc, w2, zero, lut):
    mesh = plsc.VectorSubcoreMesh(core_axis_name="c", subcore_axis_name="s",
                                  num_cores=NC, num_subcores=NS)
    kern = pl.kernel(
        _edge_body,
        out_type=jax.ShapeDtypeStruct((NW, 3 * N_NODES), jnp.float32),
        mesh=mesh,
        compiler_params=pltpu.CompilerParams(needs_layout_passes=False),
        scratch_types=[
            pltpu.VMEM((3 * N_NODES,), jnp.float32),
            pltpu.VMEM((DIM * 16,), jnp.float32),
            pltpu.VMEM((DIM * 16,), jnp.float32),
            pltpu.VMEM((E_PER_W,), jnp.int32),
            pltpu.VMEM((E_PER_W,), jnp.int32),
            pltpu.VMEM((LUT_N,), jnp.float32),
            pltpu.VMEM((G, DIM), jnp.float32),
            pltpu.VMEM((G, DIM), jnp.float32),
            pltpu.VMEM((G, DIM), jnp.float32),
            pltpu.VMEM((3 * N_NODES,), jnp.float32),
            pltpu.SemaphoreType.DMA,
            pltpu.SemaphoreType.DMA,
            pltpu.SemaphoreType.DMA,
            pltpu.SemaphoreType.DMA,
            pltpu.SemaphoreType.DMA,
            pltpu.SemaphoreType.DMA,
        ],
    )
    return kern(a, b, x_flat, row, col, wc, w2, zero, lut)


def kernel(h, x, edge_index, W1, b1, W2):
    row = edge_index[0].astype(jnp.int32)
    col = edge_index[1].astype(jnp.int32)
    wa_t = W1[:, :DIM].T * LUT_SCALE
    wb_t = W1[:, DIM:2 * DIM].T * LUT_SCALE
    b1s = b1 * LUT_SCALE + LUT_OFF * 0.5
    wc = W1[:, 2 * DIM] * LUT_SCALE
    w2 = W2.reshape(DIM) * (1.0 / LUT_SCALE)
    rot = (jnp.arange(DIM)[:, None] + jnp.arange(16)[None, :]) & (DIM - 1)
    wc = wc[rot].reshape(DIM * 16)
    w2 = w2[rot].reshape(DIM * 16)
    x_flat = x.reshape(-1)
    zero = jnp.zeros((3 * N_NODES,), jnp.float32)
    lut = jax.nn.sigmoid(
        (jnp.arange(LUT_N, dtype=jnp.float32) - LUT_OFF) * (1.0 / LUT_SCALE))

    a, b = _node_proj(h, wa_t, wb_t, b1s)
    partials = _edge_pass(a, b, x_flat, row, col, wc, w2, zero, lut)
    v = _reduce_partials(partials)
    return v.reshape(N_NODES, 3)


# speedup vs baseline: 12.9496x; 1.1423x over previous
import functools

import jax
import jax.numpy as jnp
from jax import lax
from jax.experimental import pallas as pl
from jax.experimental.pallas import tpu as pltpu
from jax.experimental.pallas import tpu_sc as plsc

DIM = 128
K = 32
WTEMP = 10.0
N_NODES = 10000
N_EDGES = 320000

NC, NS, L = 2, 16, 16
NW = NC * NS
E_PER_W = N_EDGES // NW
G = 80
NSUB = G // L
NGROUPS = E_PER_W // G


def _proj_body(h_ref, wa_ref, wb_ref, b_ref, a_ref, bo_ref):
    hblk = h_ref[...]
    a_ref[...] = jnp.dot(hblk, wa_ref[...],
                         preferred_element_type=jnp.float32) + b_ref[...]
    bo_ref[...] = jnp.dot(hblk, wb_ref[...],
                          preferred_element_type=jnp.float32) + LUT_OFF * 0.5


def _node_proj(h, wa_t, wb_t, b1):
    BM = 2000
    grid = (N_NODES // BM,)
    return pl.pallas_call(
        _proj_body,
        grid=grid,
        in_specs=[
            pl.BlockSpec((BM, DIM), lambda i: (i, 0)),
            pl.BlockSpec((DIM, DIM), lambda i: (0, 0)),
            pl.BlockSpec((DIM, DIM), lambda i: (0, 0)),
            pl.BlockSpec((1, DIM), lambda i: (0, 0)),
        ],
        out_specs=[
            pl.BlockSpec((BM, DIM), lambda i: (i, 0)),
            pl.BlockSpec((BM, DIM), lambda i: (i, 0)),
        ],
        out_shape=[
            jax.ShapeDtypeStruct((N_NODES, DIM), jnp.float32),
            jax.ShapeDtypeStruct((N_NODES, DIM), jnp.float32),
        ],
    )(h, wa_t, wb_t, b1.reshape(1, DIM))


def _reduce_body(p_ref, o_ref):
    o_ref[...] = jnp.sum(p_ref[...], axis=0, keepdims=True) * (1.0 / K)


def _reduce_partials(partials):
    return pl.pallas_call(
        _reduce_body,
        out_shape=jax.ShapeDtypeStruct((1, 3 * N_NODES), jnp.float32),
    )(partials)


def _rsqrt_f32(a):
    i = plsc.bitcast(a, jnp.int32)
    i = jnp.int32(0x5F3759DF) - lax.shift_right_arithmetic(i, 1)
    y = plsc.bitcast(i, jnp.float32)
    h = a * 0.5
    for _ in range(3):
        y = y * (1.5 - h * y * y)
    return y


LUT_N = 4096
LUT_SCALE = 128.0
LUT_OFF = float(LUT_N // 2)


def _edge_body(a_hbm, b_hbm, x_hbm, row_hbm, col_hbm, wc_hbm, w2_hbm,
               zero_hbm, lut_hbm, out_hbm,
               x_v, wc_v, w2_v, row_v, col_v, lut_v, zb0, zb1, zb2, vacc,
               sa0, sa1, sa2, sb0, sb1, sb2):
    cid = lax.axis_index("c")
    sid = lax.axis_index("s")
    wid = sid * NC + cid
    tile_base = wid * E_PER_W

    pltpu.sync_copy(x_hbm, x_v)
    pltpu.sync_copy(wc_hbm, wc_v)
    pltpu.sync_copy(w2_hbm, w2_v)
    pltpu.sync_copy(row_hbm.at[pl.ds(tile_base, E_PER_W)], row_v)
    pltpu.sync_copy(col_hbm.at[pl.ds(tile_base, E_PER_W)], col_v)
    pltpu.sync_copy(lut_hbm, lut_v)
    pltpu.sync_copy(zero_hbm, vacc)

    lanes = [lax.iota(jnp.int32, 16) + (16 * s) for s in range(NSUB)]
    bufs = (zb0, zb1, zb2)
    sas = (sa0, sa1, sa2)
    sbs = (sb0, sb1, sb2)

    def issue_a(g, p):
        pltpu.async_copy(a_hbm.at[row_v.at[pl.ds(g * G, G)]], bufs[p],
                         sas[p])

    def issue_b(g, p):
        pltpu.async_copy(b_hbm.at[col_v.at[pl.ds(g * G, G)]], bufs[p],
                         sbs[p], add=True)

    def wait(sem, buf):
        pltpu.make_async_copy(a_hbm.at[pl.ds(0, G)], buf, sem).wait()

    issue_a(0, 0)
    wait(sas[0], bufs[0])
    issue_b(0, 0)
    issue_a(1, 1)

    def process(g, p):
        @pl.when(g + 1 < NGROUPS)
        def _():
            wait(sas[(p + 1) % 3], bufs[(p + 1) % 3])
            issue_b(g + 1, (p + 1) % 3)

        @pl.when(g + 2 < NGROUPS)
        def _():
            issue_a(g + 2, (p + 2) % 3)

        r3 = []
        dsqs = []
        rijs = []
        for s in range(NSUB):
            row_s = row_v[pl.ds(g * G + 16 * s, 16)]
            col_s = col_v[pl.ds(g * G + 16 * s, 16)]
            r3_s = row_s * 3
            c3_s = col_s * 3
            dsq = jnp.zeros((16,), jnp.float32)
            rij_s = []
            for c in range(3):
                rij = (plsc.load_gather(x_v, [r3_s + c])
                       - plsc.load_gather(x_v, [c3_s + c]))
                dsq = dsq + rij * rij
                rij_s.append(rij)
            r3.append(r3_s)
            dsqs.append(dsq)
            rijs.append(rij_s)

        wait(sbs[p], bufs[p])
        zbuf = bufs[p]

        iota16 = lax.iota(jnp.int32, 16)

        def j_body(jo, accs):
            jbase = jo * 16
            out = list(accs)
            for ji in range(16):
                j = jbase + ji
                wcv = wc_v[pl.ds(j, 16)]
                w2v = w2_v[pl.ds(j, 16)]
                cid = (iota16 + j) & (DIM - 1)
                for s in range(NSUB):
                    t = (plsc.load_gather(zbuf, [lanes[s], cid])
                         + dsqs[s] * wcv)
                    tc = jnp.minimum(jnp.maximum(t, 0.0), float(LUT_N - 1))
                    sg = plsc.load_gather(lut_v, [tc.astype(jnp.int32)])
                    out[s] = out[s] + w2v * ((t - LUT_OFF) * sg)
            return tuple(out)

        accs = lax.fori_loop(0, DIM // 16, j_body,
                             tuple(jnp.zeros((16,), jnp.float32)
                                   for _ in range(NSUB)))

        for s in range(NSUB):
            u = accs[s] * (1.0 / WTEMP)
            t = jnp.exp(-2.0 * jnp.abs(u))
            th = (1.0 - t) / (1.0 + t)
            w = jnp.where(u < 0.0, -th, th)
            rs = _rsqrt_f32(dsqs[s] + 1e-8)
            wrs = w * rs
            for c in range(3):
                plsc.addupdate_scatter(vacc, [r3[s] + c], wrs * rijs[s][c])

    def group_body(g3, _):
        for p in range(3):
            g = g3 * 3 + p

            @pl.when(g < NGROUPS)
            def _():
                process(g, p)
        return 0

    lax.fori_loop(0, (NGROUPS + 2) // 3, group_body, 0)
    pltpu.sync_copy(vacc, out_hbm.at[wid])


def _edge_pass(a, b, x_flat, row, col, wc, w2, zero, lut):
    mesh = plsc.VectorSubcoreMesh(core_axis_name="c", subcore_axis_name="s",
                                  num_cores=NC, num_subcores=NS)
    kern = pl.kernel(
        _edge_body,
        out_type=jax.ShapeDtypeStruct((NW, 3 * N_NODES), jnp.float32),
        mesh=mesh,
        compiler_params=pltpu.CompilerParams(needs_layout_passes=False),
        scratch_types=[
            pltpu.VMEM((3 * N_NODES,), jnp.float32),
            pltpu.VMEM((2 * DIM,), jnp.float32),
            pltpu.VMEM((2 * DIM,), jnp.float32),
            pltpu.VMEM((E_PER_W,), jnp.int32),
            pltpu.VMEM((E_PER_W,), jnp.int32),
            pltpu.VMEM((LUT_N,), jnp.float32),
            pltpu.VMEM((G, DIM), jnp.float32),
            pltpu.VMEM((G, DIM), jnp.float32),
            pltpu.VMEM((G, DIM), jnp.float32),
            pltpu.VMEM((3 * N_NODES,), jnp.float32),
            pltpu.SemaphoreType.DMA,
            pltpu.SemaphoreType.DMA,
            pltpu.SemaphoreType.DMA,
            pltpu.SemaphoreType.DMA,
            pltpu.SemaphoreType.DMA,
            pltpu.SemaphoreType.DMA,
        ],
    )
    return kern(a, b, x_flat, row, col, wc, w2, zero, lut)


def kernel(h, x, edge_index, W1, b1, W2):
    row = edge_index[0].astype(jnp.int32)
    col = edge_index[1].astype(jnp.int32)
    wa_t = W1[:, :DIM].T * LUT_SCALE
    wb_t = W1[:, DIM:2 * DIM].T * LUT_SCALE
    b1s = b1 * LUT_SCALE + LUT_OFF * 0.5
    wc = W1[:, 2 * DIM] * LUT_SCALE
    w2 = W2.reshape(DIM) * (1.0 / LUT_SCALE)
    wc = jnp.concatenate([wc, wc])
    w2 = jnp.concatenate([w2, w2])
    x_flat = x.reshape(-1)
    zero = jnp.zeros((3 * N_NODES,), jnp.float32)
    lut = jax.nn.sigmoid(
        (jnp.arange(LUT_N, dtype=jnp.float32) - LUT_OFF) * (1.0 / LUT_SCALE))

    a, b = _node_proj(h, wa_t, wb_t, b1s)
    partials = _edge_pass(a, b, x_flat, row, col, wc, w2, zero, lut)
    v = _reduce_partials(partials)
    return v.reshape(N_NODES, 3)
